# baseline (speedup 1.0000x reference)
import functools

import jax
import jax.numpy as jnp
from jax import lax
from jax.experimental import pallas as pl
from jax.experimental.pallas import tpu as pltpu
from jax.experimental.pallas import tpu_sc as plsc

CTX = 200
EMBED = 64
VOCAB = 1000000

_CHUNK_A = 128
_CHUNK_B = CTX - _CHUNK_A


def _sc_gather_sum_body(idx_hbm, table_hbm, out_hbm,
                        idx_a, idx_b, rows_a, rows_b, acc_v, sem):
    c = lax.axis_index("c")
    s = lax.axis_index("s")

    @pl.when(jnp.logical_and(c == 0, s == 0))
    def _():
        pltpu.sync_copy(idx_hbm.at[pl.ds(0, _CHUNK_A)], idx_a)
        pltpu.sync_copy(idx_hbm.at[pl.ds(_CHUNK_A, _CHUNK_B)], idx_b)
        cp_a = pltpu.async_copy(table_hbm.at[idx_a], rows_a, sem)
        cp_b = pltpu.async_copy(table_hbm.at[idx_b], rows_b, sem)
        cp_a.wait()
        cp_b.wait()

        def accum(rows_ref, n, carry):
            def body(i, acc):
                return tuple(
                    acc[k] + rows_ref[i, pl.ds(k * 16, 16)] for k in range(4)
                )
            return lax.fori_loop(0, n, body, carry)

        zero = jnp.zeros((16,), jnp.float32)
        acc = accum(rows_a, _CHUNK_A, (zero, zero, zero, zero))
        acc = accum(rows_b, _CHUNK_B, acc)
        for k in range(4):
            acc_v[pl.ds(k * 16, 16)] = acc[k]
        pltpu.sync_copy(acc_v, out_hbm)


@jax.jit
def _sc_gather_sum(inputs, emb_table):
    mesh = plsc.VectorSubcoreMesh(core_axis_name="c", subcore_axis_name="s")
    return pl.kernel(
        _sc_gather_sum_body,
        out_type=jax.ShapeDtypeStruct((EMBED,), jnp.float32),
        mesh=mesh,
        scratch_types=[
            pltpu.VMEM((_CHUNK_A,), jnp.int32),
            pltpu.VMEM((_CHUNK_B,), jnp.int32),
            pltpu.VMEM((_CHUNK_A, EMBED), jnp.float32),
            pltpu.VMEM((_CHUNK_B, EMBED), jnp.float32),
            pltpu.VMEM((EMBED,), jnp.float32),
            pltpu.SemaphoreType.DMA,
        ],
        compiler_params=pltpu.CompilerParams(use_tc_tiling_on_sc=False),
    )(inputs, emb_table)


_BLK = 16384


def _tc_matvec_body(e_ref, w_ref, b_ref, o_ref):
    e = e_ref[:]
    w = w_ref[:]
    acc = lax.dot_general(
        e, w, (((1,), (1,)), ((), ())), preferred_element_type=jnp.float32
    )
    o_ref[:] = acc[0] + b_ref[:]


@jax.jit
def _tc_matvec(e8, W, b):
    nblk = pl.cdiv(VOCAB, _BLK)
    return pl.pallas_call(
        _tc_matvec_body,
        grid=(nblk,),
        in_specs=[
            pl.BlockSpec((8, EMBED), lambda i: (0, 0)),
            pl.BlockSpec((_BLK, EMBED), lambda i: (i, 0)),
            pl.BlockSpec((_BLK,), lambda i: (i,)),
        ],
        out_specs=pl.BlockSpec((_BLK,), lambda i: (i,)),
        out_shape=jax.ShapeDtypeStruct((VOCAB,), jnp.float32),
    )(e8, W, b)


def kernel(inputs, emb_table, W, b):
    embeds = _sc_gather_sum(inputs, emb_table)
    e8 = jnp.broadcast_to(embeds.reshape(1, EMBED), (8, EMBED))
    return _tc_matvec(e8, W, b)


# speedup vs baseline: 6.8584x; 6.8584x over previous
import functools

import jax
import jax.numpy as jnp
from jax import lax
from jax.experimental import pallas as pl
from jax.experimental.pallas import tpu as pltpu
from jax.experimental.pallas import tpu_sc as plsc

CTX = 200
EMBED = 64
VOCAB = 1000000

_RING = 8
_GROUPS = CTX // _RING
_VLAST = VOCAB - 128


def _sc_gather_sum_body(idx_hbm, tabt_hbm, out_hbm, idx_vm, blks, acc_v, *sems):
    first = jnp.logical_and(lax.axis_index("c") == 0, lax.axis_index("s") == 0)

    @pl.when(first)
    def _():
        pltpu.sync_copy(idx_hbm, idx_vm.at[pl.ds(0, CTX)])

        def get_v(i):
            base = (i // 16) * 16
            chunk = idx_vm[pl.ds(base, 16)]
            sel = lax.iota(jnp.int32, 16) == (i - base)
            return jnp.sum(jnp.where(sel, chunk, 0))

        def col_base(v):
            return pl.multiple_of(v - (v & 127), 128)

        def issue(i, b):
            pltpu.make_async_copy(
                tabt_hbm.at[:, pl.ds(col_base(get_v(i)), 128)], blks.at[b],
                sems[b]
            ).start()

        for b in range(_RING):
            issue(b, b)

        def group(g, acc):
            for b in range(_RING):
                i = g * _RING + b
                pltpu.make_async_copy(
                    tabt_hbm.at[:, pl.ds(0, 128)], blks.at[b], sems[b]
                ).wait()
                v = get_v(i)
                o = v - col_base(v)
                cols = jnp.full((16,), o, jnp.int32)
                new = []
                for k in range(4):
                    rows = lax.iota(jnp.int32, 16) + 16 * k
                    new.append(acc[k] + plsc.load_gather(blks.at[b], [rows, cols]))

                @pl.when(i + _RING < CTX)
                def _():
                    issue(i + _RING, b)

                acc = tuple(new)
            return acc

        zero = jnp.zeros((16,), jnp.float32)
        acc = lax.fori_loop(0, _GROUPS, group, (zero,) * 4)
        for k in range(4):
            acc_v[pl.ds(16 * k, 16)] = acc[k]
        pltpu.sync_copy(acc_v, out_hbm)


@jax.jit
def _sc_gather_sum(inputs, tab_t):
    mesh = plsc.VectorSubcoreMesh(core_axis_name="c", subcore_axis_name="s")
    return pl.kernel(
        _sc_gather_sum_body,
        out_type=jax.ShapeDtypeStruct((EMBED,), jnp.float32),
        mesh=mesh,
        scratch_types=[
            pltpu.VMEM((208,), jnp.int32),
            pltpu.VMEM((_RING, EMBED, 128), jnp.float32),
            pltpu.VMEM((EMBED,), jnp.float32),
        ] + [pltpu.SemaphoreType.DMA] * _RING,
        compiler_params=pltpu.CompilerParams(needs_layout_passes=False),
    )(inputs, tab_t)


_BLKV = 32768


def _tc_matvec_body(e_ref, wt_ref, b_ref, o_ref):
    acc = lax.dot_general(
        e_ref[:], wt_ref[:], (((1,), (0,)), ((), ())),
        preferred_element_type=jnp.float32,
    )
    o_ref[:] = acc[0] + b_ref[:]


@jax.jit
def _tc_matvec(e8, W_t, b):
    nblk = pl.cdiv(VOCAB, _BLKV)
    return pl.pallas_call(
        _tc_matvec_body,
        grid=(nblk,),
        in_specs=[
            pl.BlockSpec((8, EMBED), lambda i: (0, 0)),
            pl.BlockSpec((EMBED, _BLKV), lambda i: (0, i)),
            pl.BlockSpec((_BLKV,), lambda i: (i,)),
        ],
        out_specs=pl.BlockSpec((_BLKV,), lambda i: (i,)),
        out_shape=jax.ShapeDtypeStruct((VOCAB,), jnp.float32),
    )(e8, W_t, b)


def kernel(inputs, emb_table, W, b):
    embeds = _sc_gather_sum(inputs, emb_table.T)
    e8 = jnp.broadcast_to(embeds.reshape(1, EMBED), (8, EMBED))
    return _tc_matvec(e8, W.T, b)


# speedup vs baseline: 10.5323x; 1.5357x over previous
import functools

import jax
import jax.numpy as jnp
from jax import lax
from jax.experimental import pallas as pl
from jax.experimental.pallas import tpu as pltpu
from jax.experimental.pallas import tpu_sc as plsc

CTX = 200
EMBED = 64
VOCAB = 1000000

_RING = 4
_NW = 32
_SLOTS = (CTX + _NW - 1) // _NW


def _sc_gather_sum_body(idx_hbm, tabt_hbm, out_hbm, idx_vm, blks, acc_v, *sems):
    c = lax.axis_index("c")
    t = lax.axis_index("s")
    wid = c * 16 + t

    pltpu.sync_copy(idx_hbm, idx_vm.at[pl.ds(0, CTX)])

    def get_v(j):
        chunk = idx_vm[pl.ds(32 * j + 16 * c, 16)]
        sel = lax.iota(jnp.int32, 16) == t
        return jnp.sum(jnp.where(sel, chunk, 0))

    def col_base(v):
        return pl.multiple_of(v - (v & 127), 128)

    def issue(j, b):
        pltpu.make_async_copy(
            tabt_hbm.at[:, pl.ds(col_base(get_v(j)), 128)], blks.at[b], sems[b]
        ).start()

    def active(j):
        return (wid + 32 * j < CTX) if 32 * j + 31 >= CTX else None

    def when_active(j, fn):
        a = active(j)
        if a is None:
            fn()
        else:
            pl.when(a)(fn)

    for b in range(min(_RING, _SLOTS)):
        when_active(b, lambda b=b: issue(b, b))

    acc = [jnp.zeros((16,), jnp.float32) for _ in range(4)]
    for j in range(_SLOTS):
        b = j % _RING
        when_active(j, lambda b=b: pltpu.make_async_copy(
            tabt_hbm.at[:, pl.ds(0, 128)], blks.at[b], sems[b]).wait())
        v = get_v(j)
        o = v - col_base(v)
        cols = jnp.full((16,), o, jnp.int32)
        a = active(j)
        for k in range(4):
            rows = lax.iota(jnp.int32, 16) + 16 * k
            g = plsc.load_gather(blks.at[b], [rows, cols])
            acc[k] = acc[k] + g if a is None else acc[k] + jnp.where(a, g, 0.0)
        if j + _RING < _SLOTS:
            when_active(j + _RING, lambda j=j, b=b: issue(j + _RING, b))

    for k in range(4):
        acc_v[k, :] = acc[k]
    pltpu.sync_copy(acc_v, out_hbm.at[wid])


@jax.jit
def _sc_gather_sum(inputs, tab_t):
    mesh = plsc.VectorSubcoreMesh(core_axis_name="c", subcore_axis_name="s")
    return pl.kernel(
        _sc_gather_sum_body,
        out_type=jax.ShapeDtypeStruct((_NW, 4, 16), jnp.float32),
        mesh=mesh,
        scratch_types=[
            pltpu.VMEM((32 * _SLOTS, ), jnp.int32),
            pltpu.VMEM((_RING, EMBED, 128), jnp.float32),
            pltpu.VMEM((4, 16), jnp.float32),
        ] + [pltpu.SemaphoreType.DMA] * _RING,
        compiler_params=pltpu.CompilerParams(needs_layout_passes=False),
    )(inputs, tab_t)


_BLKV = 32768


def _tc_matvec_body(p_ref, wt_ref, b_ref, o_ref):
    e = jnp.sum(p_ref[:], axis=0, keepdims=True)
    e8 = jnp.broadcast_to(e, (8, EMBED))
    acc = lax.dot_general(
        e8, wt_ref[:], (((1,), (0,)), ((), ())),
        preferred_element_type=jnp.float32,
    )
    o_ref[:] = acc[0] + b_ref[:]


@jax.jit
def _tc_matvec(partials, W_t, b):
    nblk = pl.cdiv(VOCAB, _BLKV)
    return pl.pallas_call(
        _tc_matvec_body,
        grid=(nblk,),
        in_specs=[
            pl.BlockSpec((32, EMBED), lambda i: (0, 0)),
            pl.BlockSpec((EMBED, _BLKV), lambda i: (0, i)),
            pl.BlockSpec((_BLKV,), lambda i: (i,)),
        ],
        out_specs=pl.BlockSpec((_BLKV,), lambda i: (i,)),
        out_shape=jax.ShapeDtypeStruct((VOCAB,), jnp.float32),
    )(partials, W_t, b)


def kernel(inputs, emb_table, W, b):
    partials = _sc_gather_sum(inputs, emb_table.T)
    return _tc_matvec(partials.reshape(_NW, EMBED), W.T, b)
